# baseline (speedup 1.0000x reference)
import functools

import jax
import jax.numpy as jnp
from jax import lax
from jax.experimental import pallas as pl
from jax.experimental.pallas import tpu as pltpu
from jax.experimental.pallas import tpu_sc as plsc

N = 100000
DIM_H = 512
NUM_GRAPHS = 256
ROWS = 1000
NB = N // ROWS

NW = 32
CHUNK = 128
NCHUNK = -(-N // CHUNK)
LAST_START = N - CHUNK
TRIPS = -(-NCHUNK // NW)


def _pool_ffn_body(batch_ref, h_ref, W1_ref, b1_ref, W2_ref, b2_ref,
                   out_ref, acc_ref):
    i = pl.program_id(0)

    @pl.when(i == 0)
    def _init():
        acc_ref[...] = jnp.zeros_like(acc_ref)

    ids = batch_ref[0, 0, :]
    seg = lax.broadcasted_iota(jnp.int32, (NUM_GRAPHS, ROWS), 0)
    onehot = (ids[None, :] == seg).astype(jnp.float32)
    acc_ref[...] += jnp.dot(onehot, h_ref[...],
                            preferred_element_type=jnp.float32)

    @pl.when(i == NB - 1)
    def _ffn():
        s = acc_ref[...]
        z = jnp.maximum(jnp.dot(s, W1_ref[...],
                                preferred_element_type=jnp.float32)
                        + b1_ref[...], 0.0)
        out_ref[...] = jnp.dot(z, W2_ref[...],
                               preferred_element_type=jnp.float32) + b2_ref[...]


def _sc_broadcast_body(table_hbm, idx_hbm, out_hbm, idx_v, rows_v, sem):
    cid = lax.axis_index("c")
    sid = lax.axis_index("s")
    wid = cid * 16 + sid

    def body(t, carry):
        g = wid + t * NW

        @pl.when(g < NCHUNK)
        def _do():
            start = jnp.where(g == NCHUNK - 1, LAST_START, g * CHUNK)
            pltpu.sync_copy(idx_hbm.at[pl.ds(start, CHUNK)], idx_v)
            pltpu.async_copy(table_hbm.at[idx_v], rows_v, sem).wait()
            pltpu.sync_copy(rows_v, out_hbm.at[pl.ds(start, CHUNK)])

        return carry

    lax.fori_loop(0, TRIPS, body, 0)


@jax.jit
def kernel(h, batch, W1, b1, W2, b2):
    batch_i32 = batch.astype(jnp.int32)
    batch3 = batch_i32.reshape(NB, 1, ROWS)

    h_vn = pl.pallas_call(
        _pool_ffn_body,
        grid=(NB,),
        in_specs=[
            pl.BlockSpec((1, 1, ROWS), lambda i: (i, 0, 0)),
            pl.BlockSpec((ROWS, DIM_H), lambda i: (i, 0)),
            pl.BlockSpec((DIM_H, 2 * DIM_H), lambda i: (0, 0)),
            pl.BlockSpec((2 * DIM_H,), lambda i: (0,)),
            pl.BlockSpec((2 * DIM_H, DIM_H), lambda i: (0, 0)),
            pl.BlockSpec((DIM_H,), lambda i: (0,)),
        ],
        out_specs=pl.BlockSpec((NUM_GRAPHS, DIM_H), lambda i: (0, 0)),
        out_shape=jax.ShapeDtypeStruct((NUM_GRAPHS, DIM_H), jnp.float32),
        scratch_shapes=[pltpu.VMEM((NUM_GRAPHS, DIM_H), jnp.float32)],
    )(batch3, h, W1, b1, W2, b2)

    sc_gather = pl.kernel(
        _sc_broadcast_body,
        out_type=jax.ShapeDtypeStruct((N, DIM_H), jnp.float32),
        mesh=plsc.VectorSubcoreMesh(core_axis_name="c", subcore_axis_name="s"),
        scratch_types=[
            pltpu.VMEM((CHUNK,), jnp.int32),
            pltpu.VMEM((CHUNK, DIM_H), jnp.float32),
            pltpu.SemaphoreType.DMA,
        ],
    )
    return sc_gather(h_vn, batch_i32)
